# Initial kernel scaffold; baseline (speedup 1.0000x reference)
import jax
import jax.numpy as jnp
from jax.experimental import pallas as pl


def kernel(x_n0, x_n1, edge_index_n0_n0, edge_index_n0_n1, edge_index_n1_n0, Wsrc, bsrc, Wdst, bdst, Wupd, bupd, bn_gamma, bn_beta, post_W, post_b):
    raise NotImplementedError("write your pallas kernel here")


# speedup vs baseline: 1.8635x; 1.8635x over previous
import jax
import jax.numpy as jnp
from jax import lax
from jax.experimental import pallas as pl
from jax.experimental.pallas import tpu as pltpu
from jax.experimental.pallas import tpu_sc as plsc

N = 50000
E = 400000
D = 64
Q = 16
C = 128
NCH = E // C
NSUB = 16
NCORE = 2
CPT = 10
RPT = N // CPT
ZR = 1000

_MESH = plsc.VectorSubcoreMesh(
    core_axis_name="c", subcore_axis_name="s", num_cores=NCORE,
    num_subcores=NSUB)

_SC_PARAMS = pltpu.CompilerParams(use_tc_tiling_on_sc=False)

_f32 = jnp.float32


def _zero_fill(buf):
  z16 = jnp.zeros((16,), _f32)

  def body(i, _):
    buf[i, :] = z16
    return 0

  lax.fori_loop(0, buf.shape[0], body, 0)


def _sums_body(t0, t1, t2, t3, src_hbm, dst_hbm, o0, o1, o2, o3,
               src_v, dst_v, rows_v, zbuf, acc, sem):
  c = lax.axis_index("c")
  s = lax.axis_index("s")

  _zero_fill(zbuf)

  q, r = NCH // NSUB, NCH % NSUB
  n_my = jnp.where(s < r, q + 1, q)
  start = s * q + jnp.minimum(s, r)

  for (ta, tb, oa, ob) in ((t0, t1, o0, o1), (t2, t3, o2, o3)):
    @pl.when(s < CPT)
    def _():
      for k in range(RPT // ZR):
        pltpu.sync_copy(zbuf, acc.at[pl.ds(s * RPT + k * ZR, ZR)])
    plsc.subcore_barrier()

    def chunk(k, _):
      @pl.when(k < n_my)
      def _():
        base = (start + k) * C
        pltpu.sync_copy(src_hbm.at[pl.ds(base, C)], src_v)
        pltpu.sync_copy(dst_hbm.at[pl.ds(base, C)], dst_v)

        @pl.when(c == 0)
        def _():
          pltpu.async_copy(ta.at[src_v], rows_v, sem).wait()

        @pl.when(c == 1)
        def _():
          pltpu.async_copy(tb.at[src_v], rows_v, sem).wait()

        pltpu.sync_copy(rows_v, acc.at[dst_v], add=True)
      return 0

    lax.fori_loop(0, q + 1, chunk, 0)
    plsc.subcore_barrier()

    @pl.when(jnp.logical_and(c == 0, s < CPT))
    def _():
      pltpu.sync_copy(acc.at[pl.ds(s * RPT, RPT)], oa.at[pl.ds(s * RPT, RPT)])

    @pl.when(jnp.logical_and(c == 1, s < CPT))
    def _():
      pltpu.sync_copy(acc.at[pl.ds(s * RPT, RPT)], ob.at[pl.ds(s * RPT, RPT)])
    plsc.subcore_barrier()


_sc_sums = pl.kernel(
    _sums_body,
    out_type=tuple(jax.ShapeDtypeStruct((N, Q), _f32) for _ in range(4)),
    mesh=_MESH,
    scratch_types=[
        pltpu.VMEM((C,), jnp.int32),
        pltpu.VMEM((C,), jnp.int32),
        pltpu.VMEM((C, Q), _f32),
        pltpu.VMEM((ZR, Q), _f32),
        pltpu.VMEM_SHARED((N, Q), _f32),
        pltpu.SemaphoreType.DMA,
    ],
    compiler_params=_SC_PARAMS,
    name="sc_segment_sums")


def _counts_body(dst0, dst1, dst2, o0lo, o0hi, o1lo, o1hi, o2lo, o2hi,
                 dst_v, ones_v, zbuf, acc):
  c = lax.axis_index("c")
  s = lax.axis_index("s")

  lane = lax.broadcasted_iota(jnp.int32, (16,), 0)
  one_row = jnp.where(lane == 0, 1.0, 0.0).astype(_f32)

  def fill(i, _):
    ones_v[i, :] = one_row
    return 0

  lax.fori_loop(0, C, fill, 0)
  _zero_fill(zbuf)

  half = NCH // 2 + 1
  t_c = jnp.where(c == 0, half, NCH - half)
  base_c = c * half

  q = (half + NSUB - 1) // NSUB
  n_my = jnp.minimum(jnp.maximum(t_c - s * q, 0), q)
  start = base_c + s * q

  for (dst_e, olo, ohi) in ((dst0, o0lo, o0hi), (dst1, o1lo, o1hi),
                            (dst2, o2lo, o2hi)):
    @pl.when(s < CPT)
    def _():
      for k in range(RPT // ZR):
        pltpu.sync_copy(zbuf, acc.at[pl.ds(s * RPT + k * ZR, ZR)])
    plsc.subcore_barrier()

    def chunk(k, _):
      @pl.when(k < n_my)
      def _():
        base = (start + k) * C
        pltpu.sync_copy(dst_e.at[pl.ds(base, C)], dst_v)
        pltpu.sync_copy(ones_v, acc.at[dst_v], add=True)
      return 0

    lax.fori_loop(0, q, chunk, 0)
    plsc.subcore_barrier()

    @pl.when(jnp.logical_and(c == 0, s < CPT))
    def _():
      pltpu.sync_copy(acc.at[pl.ds(s * RPT, RPT)], olo.at[pl.ds(s * RPT, RPT)])

    @pl.when(jnp.logical_and(c == 1, s < CPT))
    def _():
      pltpu.sync_copy(acc.at[pl.ds(s * RPT, RPT)], ohi.at[pl.ds(s * RPT, RPT)])
    plsc.subcore_barrier()


_sc_counts = pl.kernel(
    _counts_body,
    out_type=tuple(jax.ShapeDtypeStruct((N, 16), _f32) for _ in range(6)),
    mesh=_MESH,
    scratch_types=[
        pltpu.VMEM((C,), jnp.int32),
        pltpu.VMEM((C, 16), _f32),
        pltpu.VMEM((ZR, 16), _f32),
        pltpu.VMEM_SHARED((N, 16), _f32),
    ],
    compiler_params=_SC_PARAMS,
    name="sc_segment_counts")


RBLK = 2000
NBLK = N // RBLK


def _layer_mm(s00, s01, s10, rcps, h0s, h1s, w0, w1, c0, c1):
  qspec = pl.BlockSpec((RBLK, Q), lambda i: (i, 0))
  vspec = pl.BlockSpec((RBLK, 1), lambda i: (i, 0))
  wspec0 = pl.BlockSpec((192, D), lambda i: (0, 0))
  wspec1 = pl.BlockSpec((128, D), lambda i: (0, 0))
  bspec = pl.BlockSpec((1, D), lambda i: (0, 0))
  sspec = pl.BlockSpec((4, D), lambda i: (0, 0))

  def body(s00a, s00b, s00c, s00d, s01a, s01b, s01c, s01d,
           s10a, s10b, s10c, s10d, r00, r01, r10,
           h0a, h0b, h0c, h0d, h1a, h1b, h1c, h1d,
           w0r, w1r, c0r, c1r, n0_out, n1_out, stats_out, stats_acc):
    i = pl.program_id(0)
    x0 = jnp.concatenate(
        [t[...] * r00[...] for t in (s00a, s00b, s00c, s00d)]
        + [t[...] * r10[...] for t in (s10a, s10b, s10c, s10d)]
        + [t[...] for t in (h0a, h0b, h0c, h0d)], axis=1)
    n0 = jnp.dot(x0, w0r[...], preferred_element_type=_f32) + c0r[...]
    x1 = jnp.concatenate(
        [t[...] * r01[...] for t in (s01a, s01b, s01c, s01d)]
        + [t[...] for t in (h1a, h1b, h1c, h1d)], axis=1)
    n1 = jnp.dot(x1, w1r[...], preferred_element_type=_f32) + c1r[...]

    n0_out[...] = n0
    n1_out[...] = n1

    blk = jnp.stack([jnp.sum(n0, axis=0), jnp.sum(n0 * n0, axis=0),
                     jnp.sum(n1, axis=0), jnp.sum(n1 * n1, axis=0)])

    @pl.when(i == 0)
    def _():
      stats_acc[...] = blk

    @pl.when(i > 0)
    def _():
      stats_acc[...] += blk

    @pl.when(i == NBLK - 1)
    def _():
      stats_out[...] = stats_acc[...]

  return pl.pallas_call(
      body,
      grid=(NBLK,),
      in_specs=[qspec] * 12 + [vspec] * 3 + [qspec] * 8
      + [wspec0, wspec1, bspec, bspec],
      out_specs=[pl.BlockSpec((RBLK, D), lambda i: (i, 0))] * 2 + [sspec],
      out_shape=(jax.ShapeDtypeStruct((N, D), _f32),
                 jax.ShapeDtypeStruct((N, D), _f32),
                 jax.ShapeDtypeStruct((4, D), _f32)),
      scratch_shapes=[pltpu.VMEM((4, D), _f32)],
      name="tc_layer_matmuls",
  )(*s00, *s01, *s10, *rcps, *h0s, *h1s, w0, w1, c0, c1)


def _bn_act_body(n0, n1, scl0, sh0, scl1, sh1, *outs):
  a0 = n0[...] * scl0[...] + sh0[...]
  a0 = jnp.where(a0 >= 0, a0, 0.01 * a0)
  a1 = n1[...] * scl1[...] + sh1[...]
  a1 = jnp.where(a1 >= 0, a1, 0.01 * a1)
  for k in range(4):
    outs[k][...] = a0[:, k * Q:(k + 1) * Q]
    outs[4 + k][...] = a1[:, k * Q:(k + 1) * Q]


def _bn_act(n0, n1, scl0, sh0, scl1, sh1):
  nspec = pl.BlockSpec((RBLK, D), lambda i: (i, 0))
  bspec = pl.BlockSpec((1, D), lambda i: (0, 0))
  ospec = pl.BlockSpec((RBLK, Q), lambda i: (i, 0))
  return pl.pallas_call(
      _bn_act_body,
      grid=(NBLK,),
      in_specs=[nspec, nspec, bspec, bspec, bspec, bspec],
      out_specs=[ospec] * 8,
      out_shape=tuple(jax.ShapeDtypeStruct((N, Q), _f32) for _ in range(8)),
      name="tc_bn_act",
  )(n0, n1, scl0, sh0, scl1, sh1)


def _bn_act_post_body(n0, n1, scl0, sh0, scl1, sh1, pw0, pb0, pw1, pb1,
                      o0, o1):
  a0 = n0[...] * scl0[...] + sh0[...]
  a0 = jnp.where(a0 >= 0, a0, 0.01 * a0)
  a1 = n1[...] * scl1[...] + sh1[...]
  a1 = jnp.where(a1 >= 0, a1, 0.01 * a1)
  o0[...] = jnp.dot(a0, pw0[...], preferred_element_type=_f32) + pb0[...]
  o1[...] = jnp.dot(a1, pw1[...], preferred_element_type=_f32) + pb1[...]


def _bn_act_post(n0, n1, scl0, sh0, scl1, sh1, pw0, pb0, pw1, pb1):
  NL = pw0.shape[1]
  nspec = pl.BlockSpec((RBLK, D), lambda i: (i, 0))
  bspec = pl.BlockSpec((1, D), lambda i: (0, 0))
  wspec = pl.BlockSpec((D, NL), lambda i: (0, 0))
  pspec = pl.BlockSpec((1, NL), lambda i: (0, 0))
  ospec = pl.BlockSpec((RBLK, NL), lambda i: (i, 0))
  return pl.pallas_call(
      _bn_act_post_body,
      grid=(NBLK,),
      in_specs=[nspec, nspec, bspec, bspec, bspec, bspec,
                wspec, pspec, wspec, pspec],
      out_specs=[ospec, ospec],
      out_shape=(jax.ShapeDtypeStruct((N, NL), _f32),
                 jax.ShapeDtypeStruct((N, NL), _f32)),
      name="tc_bn_act_post",
  )(n0, n1, scl0, sh0, scl1, sh1, pw0, pb0, pw1, pb1)


def _scale_shift(stats, gamma, beta):
  m = stats[0] / N
  v = stats[1] / N - m * m
  scl = gamma / jnp.sqrt(v + 1.0)
  return scl[None, :], (beta - m * scl)[None, :]


def kernel(x_n0, x_n1, edge_index_n0_n0, edge_index_n0_n1, edge_index_n1_n0,
           Wsrc, bsrc, Wdst, bdst, Wupd, bupd, bn_gamma, bn_beta,
           post_W, post_b):
  ei00 = edge_index_n0_n0.astype(jnp.int32)
  ei01 = edge_index_n0_n1.astype(jnp.int32)
  ei10 = edge_index_n1_n0.astype(jnp.int32)

  cps = _sc_counts(ei00[1], ei01[1], ei10[1])
  rcps = []
  for t in range(3):
    cnt = cps[2 * t][:, 0] + cps[2 * t + 1][:, 0]
    rcps.append((1.0 / jnp.maximum(cnt, 1.0))[:, None])

  As = jnp.einsum("ltij,ltjk->ltik", Wsrc, Wupd[:, :, D:, :])
  Ad = jnp.einsum("ltij,ltjk->ltik", Wdst, Wupd[:, :, :D, :])
  cc = (jnp.einsum("lti,ltik->ltk", bsrc, Wupd[:, :, D:, :])
        + jnp.einsum("lti,ltik->ltk", bdst, Wupd[:, :, :D, :]) + bupd)

  h0s = tuple(x_n0[:, k * Q:(k + 1) * Q] for k in range(4))
  h1s = tuple(x_n1[:, k * Q:(k + 1) * Q] for k in range(4))

  for l in range(2):
    w0 = jnp.concatenate([As[l, 0] / 2.0, As[l, 2] / 2.0,
                          (Ad[l, 0] + Ad[l, 2]) / 2.0], axis=0)
    w1 = jnp.concatenate([As[l, 1], Ad[l, 1]], axis=0)
    c0 = ((cc[l, 0] + cc[l, 2]) / 2.0)[None, :]
    c1 = cc[l, 1][None, :]

    s00 = _sc_sums(*h0s, ei00[0], ei00[1])
    s01 = _sc_sums(*h0s, ei01[0], ei01[1])
    s10 = _sc_sums(*h1s, ei10[0], ei10[1])

    n0, n1, stats = _layer_mm(s00, s01, s10,
                              (rcps[0], rcps[1], rcps[2]),
                              h0s, h1s, w0, w1, c0, c1)

    scl0, sh0 = _scale_shift(stats[0:2], bn_gamma[l, 0], bn_beta[l, 0])
    scl1, sh1 = _scale_shift(stats[2:4], bn_gamma[l, 1], bn_beta[l, 1])

    if l == 0:
      hs = _bn_act(n0, n1, scl0, sh0, scl1, sh1)
      h0s, h1s = hs[:4], hs[4:]
    else:
      out0, out1 = _bn_act_post(n0, n1, scl0, sh0, scl1, sh1,
                                post_W[0], post_b[0][None, :],
                                post_W[1], post_b[1][None, :])

  return jnp.concatenate([out0, out1], axis=0)


# speedup vs baseline: 4.8145x; 2.5836x over previous
import jax
import jax.numpy as jnp
from jax import lax
from jax.experimental import pallas as pl
from jax.experimental.pallas import tpu as pltpu
from jax.experimental.pallas import tpu_sc as plsc

N = 50000
E = 400000
D = 64
Q = 16
C = 128
NCH = E // C
CS = 80
NCS = E // CS
G = 8
CT0 = 312
CT1 = 320
NSUB = 16
NCORE = 2
CPT = 10
RPT = N // CPT
ZR = 1000

_MESH = plsc.VectorSubcoreMesh(
    core_axis_name="c", subcore_axis_name="s", num_cores=NCORE,
    num_subcores=NSUB)

_SC_PARAMS = pltpu.CompilerParams(use_tc_tiling_on_sc=False)

_f32 = jnp.float32


def _zero_fill(buf):
  z16 = jnp.zeros((16,), _f32)

  def body(i, _):
    buf[i, :] = z16
    return 0

  lax.fori_loop(0, buf.shape[0], body, 0)


def _sums_body(t0, t1, t2, t3, src2d, dst2d, o0, o1, o2, o3,
               srcb, dstb, rows, zbuf, acc, gsem, ssem):
  c = lax.axis_index("c")
  s = lax.axis_index("s")

  _zero_fill(zbuf)

  n_grp = jnp.where(s < NSUB - 1, CT0 // G, CT1 // G)
  start = jnp.where(s < NSUB - 1, s * CT0, (NSUB - 1) * CT0)

  def load_idx(g_idx, par):
    row0 = start + g_idx * G
    pltpu.sync_copy(src2d.at[pl.ds(row0, G)], srcb.at[pl.ds(par * G, G)])
    pltpu.sync_copy(dst2d.at[pl.ds(row0, G)], dstb.at[pl.ds(par * G, G)])

  def do_group(g, par, ta, tb):
    sb, ob = par * G, (1 - par) * G

    @pl.when(g < n_grp)
    def _():
      for b in range(G):
        @pl.when(c == 0)
        def _():
          pltpu.async_copy(ta.at[srcb.at[sb + b]], rows.at[sb + b], gsem)

        @pl.when(c == 1)
        def _():
          pltpu.async_copy(tb.at[srcb.at[sb + b]], rows.at[sb + b], gsem)

      @pl.when(g >= 1)
      def _():
        for b in range(G):
          pltpu.make_async_copy(
              rows.at[ob + b], acc.at[dstb.at[ob + b]], ssem).wait()

      @pl.when(g + 1 < n_grp)
      def _():
        load_idx(g + 1, 1 - par)

      for b in range(G):
        pltpu.make_async_copy(
            ta.at[srcb.at[sb + b]], rows.at[sb + b], gsem).wait()
      for b in range(G):
        pltpu.async_copy(rows.at[sb + b], acc.at[dstb.at[sb + b]], ssem,
                         add=True)

  for (ta, tb, oa, ob) in ((t0, t1, o0, o1), (t2, t3, o2, o3)):
    @pl.when(s < CPT)
    def _():
      for k in range(RPT // ZR):
        pltpu.sync_copy(zbuf, acc.at[pl.ds(s * RPT + k * ZR, ZR)])
    plsc.subcore_barrier()

    load_idx(0, 0)

    def pair(g2, _):
      do_group(2 * g2, 0, ta, tb)
      do_group(2 * g2 + 1, 1, ta, tb)
      return 0

    lax.fori_loop(0, (CT1 // G) // 2, pair, 0)

    last_par = ((CT0 // G) - 1) % 2

    @pl.when(s < NSUB - 1)
    def _():
      for b in range(G):
        pltpu.make_async_copy(
            rows.at[last_par * G + b],
            acc.at[dstb.at[last_par * G + b]], ssem).wait()

    last_par1 = ((CT1 // G) - 1) % 2

    @pl.when(s == NSUB - 1)
    def _():
      for b in range(G):
        pltpu.make_async_copy(
            rows.at[last_par1 * G + b],
            acc.at[dstb.at[last_par1 * G + b]], ssem).wait()

    plsc.subcore_barrier()

    @pl.when(jnp.logical_and(c == 0, s < CPT))
    def _():
      pltpu.sync_copy(acc.at[pl.ds(s * RPT, RPT)], oa.at[pl.ds(s * RPT, RPT)])

    @pl.when(jnp.logical_and(c == 1, s < CPT))
    def _():
      pltpu.sync_copy(acc.at[pl.ds(s * RPT, RPT)], ob.at[pl.ds(s * RPT, RPT)])
    plsc.subcore_barrier()


_sc_sums = pl.kernel(
    _sums_body,
    out_type=tuple(jax.ShapeDtypeStruct((N, Q), _f32) for _ in range(4)),
    mesh=_MESH,
    scratch_types=[
        pltpu.VMEM((2 * G, CS), jnp.int32),
        pltpu.VMEM((2 * G, CS), jnp.int32),
        pltpu.VMEM((2 * G, CS, Q), _f32),
        pltpu.VMEM((ZR, Q), _f32),
        pltpu.VMEM_SHARED((N, Q), _f32),
        pltpu.SemaphoreType.DMA,
        pltpu.SemaphoreType.DMA,
    ],
    compiler_params=_SC_PARAMS,
    name="sc_segment_sums")


def _counts_body(dst0, dst1, dst2, o0lo, o0hi, o1lo, o1hi, o2lo, o2hi,
                 dst_v, ones_v, zbuf, acc):
  c = lax.axis_index("c")
  s = lax.axis_index("s")

  lane = lax.broadcasted_iota(jnp.int32, (16,), 0)
  one_row = jnp.where(lane == 0, 1.0, 0.0).astype(_f32)

  def fill(i, _):
    ones_v[i, :] = one_row
    return 0

  lax.fori_loop(0, C, fill, 0)
  _zero_fill(zbuf)

  half = NCH // 2 + 1
  t_c = jnp.where(c == 0, half, NCH - half)
  base_c = c * half

  q = (half + NSUB - 1) // NSUB
  n_my = jnp.minimum(jnp.maximum(t_c - s * q, 0), q)
  start = base_c + s * q

  for (dst_e, olo, ohi) in ((dst0, o0lo, o0hi), (dst1, o1lo, o1hi),
                            (dst2, o2lo, o2hi)):
    @pl.when(s < CPT)
    def _():
      for k in range(RPT // ZR):
        pltpu.sync_copy(zbuf, acc.at[pl.ds(s * RPT + k * ZR, ZR)])
    plsc.subcore_barrier()

    def chunk(k, _):
      @pl.when(k < n_my)
      def _():
        base = (start + k) * C
        pltpu.sync_copy(dst_e.at[pl.ds(base, C)], dst_v)
        pltpu.sync_copy(ones_v, acc.at[dst_v], add=True)
      return 0

    lax.fori_loop(0, q, chunk, 0)
    plsc.subcore_barrier()

    @pl.when(jnp.logical_and(c == 0, s < CPT))
    def _():
      pltpu.sync_copy(acc.at[pl.ds(s * RPT, RPT)], olo.at[pl.ds(s * RPT, RPT)])

    @pl.when(jnp.logical_and(c == 1, s < CPT))
    def _():
      pltpu.sync_copy(acc.at[pl.ds(s * RPT, RPT)], ohi.at[pl.ds(s * RPT, RPT)])
    plsc.subcore_barrier()


_sc_counts = pl.kernel(
    _counts_body,
    out_type=tuple(jax.ShapeDtypeStruct((N, 16), _f32) for _ in range(6)),
    mesh=_MESH,
    scratch_types=[
        pltpu.VMEM((C,), jnp.int32),
        pltpu.VMEM((C, 16), _f32),
        pltpu.VMEM((ZR, 16), _f32),
        pltpu.VMEM_SHARED((N, 16), _f32),
    ],
    compiler_params=_SC_PARAMS,
    name="sc_segment_counts")


RBLK = 2000
NBLK = N // RBLK


def _layer_mm(s00, s01, s10, rcps, h0s, h1s, w0, w1, c0, c1):
  qspec = pl.BlockSpec((RBLK, Q), lambda i: (i, 0))
  vspec = pl.BlockSpec((RBLK, 1), lambda i: (i, 0))
  wspec0 = pl.BlockSpec((192, D), lambda i: (0, 0))
  wspec1 = pl.BlockSpec((128, D), lambda i: (0, 0))
  bspec = pl.BlockSpec((1, D), lambda i: (0, 0))
  sspec = pl.BlockSpec((4, D), lambda i: (0, 0))

  def body(s00a, s00b, s00c, s00d, s01a, s01b, s01c, s01d,
           s10a, s10b, s10c, s10d, r00, r01, r10,
           h0a, h0b, h0c, h0d, h1a, h1b, h1c, h1d,
           w0r, w1r, c0r, c1r, n0_out, n1_out, stats_out, stats_acc):
    i = pl.program_id(0)
    x0 = jnp.concatenate(
        [t[...] * r00[...] for t in (s00a, s00b, s00c, s00d)]
        + [t[...] * r10[...] for t in (s10a, s10b, s10c, s10d)]
        + [t[...] for t in (h0a, h0b, h0c, h0d)], axis=1)
    n0 = jnp.dot(x0, w0r[...], preferred_element_type=_f32) + c0r[...]
    x1 = jnp.concatenate(
        [t[...] * r01[...] for t in (s01a, s01b, s01c, s01d)]
        + [t[...] for t in (h1a, h1b, h1c, h1d)], axis=1)
    n1 = jnp.dot(x1, w1r[...], preferred_element_type=_f32) + c1r[...]

    n0_out[...] = n0
    n1_out[...] = n1

    blk = jnp.stack([jnp.sum(n0, axis=0), jnp.sum(n0 * n0, axis=0),
                     jnp.sum(n1, axis=0), jnp.sum(n1 * n1, axis=0)])

    @pl.when(i == 0)
    def _():
      stats_acc[...] = blk

    @pl.when(i > 0)
    def _():
      stats_acc[...] += blk

    @pl.when(i == NBLK - 1)
    def _():
      stats_out[...] = stats_acc[...]

  return pl.pallas_call(
      body,
      grid=(NBLK,),
      in_specs=[qspec] * 12 + [vspec] * 3 + [qspec] * 8
      + [wspec0, wspec1, bspec, bspec],
      out_specs=[pl.BlockSpec((RBLK, D), lambda i: (i, 0))] * 2 + [sspec],
      out_shape=(jax.ShapeDtypeStruct((N, D), _f32),
                 jax.ShapeDtypeStruct((N, D), _f32),
                 jax.ShapeDtypeStruct((4, D), _f32)),
      scratch_shapes=[pltpu.VMEM((4, D), _f32)],
      name="tc_layer_matmuls",
  )(*s00, *s01, *s10, *rcps, *h0s, *h1s, w0, w1, c0, c1)


def _bn_act_body(n0, n1, scl0, sh0, scl1, sh1, *outs):
  a0 = n0[...] * scl0[...] + sh0[...]
  a0 = jnp.where(a0 >= 0, a0, 0.01 * a0)
  a1 = n1[...] * scl1[...] + sh1[...]
  a1 = jnp.where(a1 >= 0, a1, 0.01 * a1)
  for k in range(4):
    outs[k][...] = a0[:, k * Q:(k + 1) * Q]
    outs[4 + k][...] = a1[:, k * Q:(k + 1) * Q]


def _bn_act(n0, n1, scl0, sh0, scl1, sh1):
  nspec = pl.BlockSpec((RBLK, D), lambda i: (i, 0))
  bspec = pl.BlockSpec((1, D), lambda i: (0, 0))
  ospec = pl.BlockSpec((RBLK, Q), lambda i: (i, 0))
  return pl.pallas_call(
      _bn_act_body,
      grid=(NBLK,),
      in_specs=[nspec, nspec, bspec, bspec, bspec, bspec],
      out_specs=[ospec] * 8,
      out_shape=tuple(jax.ShapeDtypeStruct((N, Q), _f32) for _ in range(8)),
      name="tc_bn_act",
  )(n0, n1, scl0, sh0, scl1, sh1)


def _bn_act_post_body(n0, n1, scl0, sh0, scl1, sh1, pw0, pb0, pw1, pb1,
                      o0, o1):
  a0 = n0[...] * scl0[...] + sh0[...]
  a0 = jnp.where(a0 >= 0, a0, 0.01 * a0)
  a1 = n1[...] * scl1[...] + sh1[...]
  a1 = jnp.where(a1 >= 0, a1, 0.01 * a1)
  o0[...] = jnp.dot(a0, pw0[...], preferred_element_type=_f32) + pb0[...]
  o1[...] = jnp.dot(a1, pw1[...], preferred_element_type=_f32) + pb1[...]


def _bn_act_post(n0, n1, scl0, sh0, scl1, sh1, pw0, pb0, pw1, pb1):
  NL = pw0.shape[1]
  nspec = pl.BlockSpec((RBLK, D), lambda i: (i, 0))
  bspec = pl.BlockSpec((1, D), lambda i: (0, 0))
  wspec = pl.BlockSpec((D, NL), lambda i: (0, 0))
  pspec = pl.BlockSpec((1, NL), lambda i: (0, 0))
  ospec = pl.BlockSpec((RBLK, NL), lambda i: (i, 0))
  return pl.pallas_call(
      _bn_act_post_body,
      grid=(NBLK,),
      in_specs=[nspec, nspec, bspec, bspec, bspec, bspec,
                wspec, pspec, wspec, pspec],
      out_specs=[ospec, ospec],
      out_shape=(jax.ShapeDtypeStruct((N, NL), _f32),
                 jax.ShapeDtypeStruct((N, NL), _f32)),
      name="tc_bn_act_post",
  )(n0, n1, scl0, sh0, scl1, sh1, pw0, pb0, pw1, pb1)


def _scale_shift(stats, gamma, beta):
  m = stats[0] / N
  v = stats[1] / N - m * m
  scl = gamma / jnp.sqrt(v + 1.0)
  return scl[None, :], (beta - m * scl)[None, :]


def kernel(x_n0, x_n1, edge_index_n0_n0, edge_index_n0_n1, edge_index_n1_n0,
           Wsrc, bsrc, Wdst, bdst, Wupd, bupd, bn_gamma, bn_beta,
           post_W, post_b):
  ei00 = edge_index_n0_n0.astype(jnp.int32)
  ei01 = edge_index_n0_n1.astype(jnp.int32)
  ei10 = edge_index_n1_n0.astype(jnp.int32)

  cps = _sc_counts(ei00[1], ei01[1], ei10[1])
  rcps = []
  for t in range(3):
    cnt = cps[2 * t][:, 0] + cps[2 * t + 1][:, 0]
    rcps.append((1.0 / jnp.maximum(cnt, 1.0))[:, None])

  As = jnp.einsum("ltij,ltjk->ltik", Wsrc, Wupd[:, :, D:, :])
  Ad = jnp.einsum("ltij,ltjk->ltik", Wdst, Wupd[:, :, :D, :])
  cc = (jnp.einsum("lti,ltik->ltk", bsrc, Wupd[:, :, D:, :])
        + jnp.einsum("lti,ltik->ltk", bdst, Wupd[:, :, :D, :]) + bupd)

  h0s = tuple(x_n0[:, k * Q:(k + 1) * Q] for k in range(4))
  h1s = tuple(x_n1[:, k * Q:(k + 1) * Q] for k in range(4))

  src00, dst00 = ei00[0].reshape(NCS, CS), ei00[1].reshape(NCS, CS)
  src01, dst01 = ei01[0].reshape(NCS, CS), ei01[1].reshape(NCS, CS)
  src10, dst10 = ei10[0].reshape(NCS, CS), ei10[1].reshape(NCS, CS)

  for l in range(2):
    w0 = jnp.concatenate([As[l, 0] / 2.0, As[l, 2] / 2.0,
                          (Ad[l, 0] + Ad[l, 2]) / 2.0], axis=0)
    w1 = jnp.concatenate([As[l, 1], Ad[l, 1]], axis=0)
    c0 = ((cc[l, 0] + cc[l, 2]) / 2.0)[None, :]
    c1 = cc[l, 1][None, :]

    s00 = _sc_sums(*h0s, src00, dst00)
    s01 = _sc_sums(*h0s, src01, dst01)
    s10 = _sc_sums(*h1s, src10, dst10)

    n0, n1, stats = _layer_mm(s00, s01, s10,
                              (rcps[0], rcps[1], rcps[2]),
                              h0s, h1s, w0, w1, c0, c1)

    scl0, sh0 = _scale_shift(stats[0:2], bn_gamma[l, 0], bn_beta[l, 0])
    scl1, sh1 = _scale_shift(stats[2:4], bn_gamma[l, 1], bn_beta[l, 1])

    if l == 0:
      hs = _bn_act(n0, n1, scl0, sh0, scl1, sh1)
      h0s, h1s = hs[:4], hs[4:]
    else:
      out0, out1 = _bn_act_post(n0, n1, scl0, sh0, scl1, sh1,
                                post_W[0], post_b[0][None, :],
                                post_W[1], post_b[1][None, :])

  return jnp.concatenate([out0, out1], axis=0)
